# Initial kernel scaffold; baseline (speedup 1.0000x reference)
import jax
import jax.numpy as jnp
from jax.experimental import pallas as pl


def kernel(inputs_label):
    raise NotImplementedError("write your pallas kernel here")


# speedup vs baseline: 1.0134x; 1.0134x over previous
import jax
import jax.numpy as jnp
from jax.experimental import pallas as pl

_DIM = 2048
_C = 7
_SEG = _DIM // _C
_ROWS = 16384
_BLK = 512
_NB = _ROWS // _BLK


def _enc_kernel(lab_ref, out_ref):
    lab = lab_ref[0, 0, :].reshape(_BLK, 1)
    start = lab * _SEG
    end = jnp.where(lab == _C - 1, _DIM, start + _SEG)
    j = jax.lax.broadcasted_iota(jnp.int32, (_BLK, _DIM), 1)
    mask = (j >= start) & (j < end)
    out_ref[...] = mask.astype(jnp.float32)


def kernel(inputs_label):
    labs = inputs_label.reshape(_NB, 1, _BLK)
    out = pl.pallas_call(
        _enc_kernel,
        grid=(_NB,),
        in_specs=[pl.BlockSpec((1, 1, _BLK), lambda i: (i, 0, 0))],
        out_specs=pl.BlockSpec((_BLK, _DIM), lambda i: (i, 0)),
        out_shape=jax.ShapeDtypeStruct((_ROWS, _DIM), jnp.float32),
    )(labs)
    return out[None]
